# baseline (speedup 1.0000x reference)
import jax
import jax.numpy as jnp
from jax.experimental import pallas as pl
from jax.experimental.pallas import tpu as pltpu

NUM_EXPERTS = 64
TOP_K = 8
BLOCK_T = 512


def _router_block(x_ref, wt_ref, logits_ref, scores_ref, ew_ref, ei_ref):
    x = x_ref[...]
    wt = wt_ref[...]
    logits = jnp.dot(x, wt, preferred_element_type=jnp.float32)
    m = jnp.max(logits, axis=-1, keepdims=True)
    e = jnp.exp(logits - m)
    scores = e / jnp.sum(e, axis=-1, keepdims=True)
    logits_ref[...] = logits
    scores_ref[...] = scores

    s = scores
    col = jax.lax.broadcasted_iota(jnp.int32, s.shape, 1)
    vals = []
    idxs = []
    for _ in range(TOP_K):
        mk = jnp.max(s, axis=-1, keepdims=True)
        ik = jnp.min(jnp.where(s == mk, col, NUM_EXPERTS), axis=-1,
                     keepdims=True)
        vals.append(mk)
        idxs.append(ik)
        s = jnp.where(col == ik, jnp.float32(-1.0), s)
    tv = jnp.concatenate(vals, axis=-1)
    ti = jnp.concatenate(idxs, axis=-1)
    ee = jnp.exp(tv - tv[:, :1])
    ew_ref[...] = ee / jnp.sum(ee, axis=-1, keepdims=True)
    ei_ref[...] = ti


def kernel(x, W):
    bs, sq, d = x.shape
    n_tok = bs * sq
    x2 = x.reshape(n_tok, d)
    wt = W.T
    grid = (n_tok // BLOCK_T,)
    logits, scores, ew, ei = pl.pallas_call(
        _router_block,
        grid=grid,
        in_specs=[
            pl.BlockSpec((BLOCK_T, d), lambda i: (i, 0)),
            pl.BlockSpec((d, NUM_EXPERTS), lambda i: (0, 0)),
        ],
        out_specs=(
            pl.BlockSpec((BLOCK_T, NUM_EXPERTS), lambda i: (i, 0)),
            pl.BlockSpec((BLOCK_T, NUM_EXPERTS), lambda i: (i, 0)),
            pl.BlockSpec((BLOCK_T, TOP_K), lambda i: (i, 0)),
            pl.BlockSpec((BLOCK_T, TOP_K), lambda i: (i, 0)),
        ),
        out_shape=(
            jax.ShapeDtypeStruct((n_tok, NUM_EXPERTS), jnp.float32),
            jax.ShapeDtypeStruct((n_tok, NUM_EXPERTS), jnp.float32),
            jax.ShapeDtypeStruct((n_tok, TOP_K), jnp.float32),
            jax.ShapeDtypeStruct((n_tok, TOP_K), jnp.int32),
        ),
    )(x2, wt)
    return scores, logits, ew, ei


# speedup vs baseline: 1.1379x; 1.1379x over previous
import jax
import jax.numpy as jnp
from jax.experimental import pallas as pl
from jax.experimental.pallas import tpu as pltpu

NUM_EXPERTS = 64
TOP_K = 8
BLOCK_T = 512


def _router_block(x_ref, wt_ref, logits_ref, scores_ref, ew_ref, ei_ref):
    x = x_ref[...]
    wt = wt_ref[...]
    logits = jnp.dot(x, wt, preferred_element_type=jnp.float32)
    m = jnp.max(logits, axis=-1, keepdims=True)
    e = jnp.exp(logits - m)
    scores = e / jnp.sum(e, axis=-1, keepdims=True)
    logits_ref[...] = logits
    scores_ref[...] = scores

    s = scores
    colf = jax.lax.broadcasted_iota(jnp.int32, s.shape, 1).astype(jnp.float32)
    big = jnp.float32(NUM_EXPERTS)
    vals = []
    idxs = []
    for _ in range(TOP_K):
        mk = jnp.max(s, axis=-1, keepdims=True)
        ik = jnp.min(jnp.where(s == mk, colf, big), axis=-1, keepdims=True)
        vals.append(mk)
        idxs.append(ik)
        s = jnp.where(colf == ik, jnp.float32(-1.0), s)
    tv = jnp.concatenate(vals, axis=-1)
    ti = jnp.concatenate(idxs, axis=-1)
    ee = jnp.exp(tv - tv[:, :1])
    ew_ref[...] = ee / jnp.sum(ee, axis=-1, keepdims=True)
    ei_ref[...] = ti.astype(jnp.int32)


def kernel(x, W):
    bs, sq, d = x.shape
    n_tok = bs * sq
    x2 = x.reshape(n_tok, d)
    wt = W.T
    grid = (n_tok // BLOCK_T,)
    logits, scores, ew, ei = pl.pallas_call(
        _router_block,
        grid=grid,
        in_specs=[
            pl.BlockSpec((BLOCK_T, d), lambda i: (i, 0)),
            pl.BlockSpec((d, NUM_EXPERTS), lambda i: (0, 0)),
        ],
        out_specs=(
            pl.BlockSpec((BLOCK_T, NUM_EXPERTS), lambda i: (i, 0)),
            pl.BlockSpec((BLOCK_T, NUM_EXPERTS), lambda i: (i, 0)),
            pl.BlockSpec((BLOCK_T, TOP_K), lambda i: (i, 0)),
            pl.BlockSpec((BLOCK_T, TOP_K), lambda i: (i, 0)),
        ),
        out_shape=(
            jax.ShapeDtypeStruct((n_tok, NUM_EXPERTS), jnp.float32),
            jax.ShapeDtypeStruct((n_tok, NUM_EXPERTS), jnp.float32),
            jax.ShapeDtypeStruct((n_tok, TOP_K), jnp.float32),
            jax.ShapeDtypeStruct((n_tok, TOP_K), jnp.int32),
        ),
    )(x2, wt)
    return scores, logits, ew, ei
